# baseline (speedup 1.0000x reference)
import functools

import jax
import jax.numpy as jnp
from jax import lax
from jax.experimental import pallas as pl
from jax.experimental.pallas import tpu as pltpu
from jax.experimental.pallas import tpu_sc as plsc

NN = 10000
EE = 320000
GG = 128
DH = 128

NC, NS = 2, 16
NW = NC * NS
CHUNK = 128
NCH0 = 80
NCH1 = 80
NCHMAX = NCH0
TOTCH = NS * (NCH0 + NCH1)
EPAD = TOTCH * CHUNK - EE
NPAD = 10240
RPS = NPAD // NS
DEGPAD = 10240

_mesh = plsc.VectorSubcoreMesh(
    core_axis_name="c", subcore_axis_name="s", num_cores=NC, num_subcores=NS)
_sc_params = pltpu.CompilerParams(needs_layout_passes=False)


@functools.partial(
    pl.kernel,
    out_type=jax.ShapeDtypeStruct((NW, DEGPAD), jnp.float32),
    mesh=_mesh,
    compiler_params=_sc_params,
    scratch_types=[
        pltpu.VMEM((NCHMAX, CHUNK), jnp.int32),
        pltpu.VMEM((DEGPAD,), jnp.float32),
    ],
)
def _deg_kernel(dst_hbm, deg_out, dst_v, deg_v):
    c = lax.axis_index("c")
    wid = c * NS + lax.axis_index("s")
    pltpu.sync_copy(dst_hbm.at[wid], dst_v)

    @pl.loop(0, DEGPAD // 16)
    def _zero(i):
        deg_v[pl.ds(i * 16, 16)] = jnp.zeros((16,), jnp.float32)

    ones = jnp.ones((16,), jnp.float32)

    def _hist(nch):
        @pl.loop(0, nch)
        def _row(j):
            @pl.loop(0, CHUNK // 16)
            def _sub(t):
                idx = dst_v[j, pl.ds(t * 16, 16)]
                plsc.addupdate_scatter(deg_v, [idx], ones)

    @pl.when(c == 0)
    def _():
        _hist(NCH0)

    @pl.when(c != 0)
    def _():
        _hist(NCH1)

    pltpu.sync_copy(deg_v, deg_out.at[wid])


@functools.partial(
    pl.kernel,
    out_type=jax.ShapeDtypeStruct((NC, NPAD, DH), jnp.float32),
    mesh=_mesh,
    compiler_params=_sc_params,
    scratch_types=[
        pltpu.VMEM((NCHMAX, CHUNK), jnp.int32),
        pltpu.VMEM((NCHMAX, CHUNK), jnp.int32),
        pltpu.VMEM((CHUNK, DH), jnp.float32),
        pltpu.VMEM_SHARED((NPAD, DH), jnp.float32),
        pltpu.SemaphoreType.DMA,
        pltpu.SemaphoreType.DMA,
        pltpu.SemaphoreType.DMA,
    ],
)
def _edge_kernel(y_hbm, src_hbm, dst_hbm, out_hbm,
                 src_v, dst_v, rows_v, agg_sh, gsem, ssem, isem):
    c = lax.axis_index("c")
    s = lax.axis_index("s")
    wid = c * NS + s
    pltpu.sync_copy(src_hbm.at[wid], src_v)
    pltpu.sync_copy(dst_hbm.at[wid], dst_v)

    @pl.loop(0, CHUNK * DH // 16)
    def _zb(i):
        r = i // (DH // 16)
        t = i % (DH // 16)
        rows_v[r, pl.ds(t * 16, 16)] = jnp.zeros((16,), jnp.float32)

    base = s * RPS

    @pl.loop(0, RPS // CHUNK)
    def _zc(i):
        pltpu.sync_copy(rows_v, agg_sh.at[pl.ds(base + i * CHUNK, CHUNK)])

    plsc.subcore_barrier()

    def _run(nch):
        @pl.loop(0, nch)
        def _chunk(j):
            pltpu.async_copy(y_hbm.at[src_v.at[j]], rows_v, gsem).wait()
            pltpu.async_copy(rows_v, agg_sh.at[dst_v.at[j]], ssem,
                             add=True).wait()

    @pl.when(c == 0)
    def _():
        _run(NCH0)

    @pl.when(c != 0)
    def _():
        _run(NCH1)

    plsc.subcore_barrier()
    pltpu.sync_copy(agg_sh.at[pl.ds(base, RPS)],
                    out_hbm.at[c, pl.ds(base, RPS)])


def _k2_body(x_ref, w_ref, degp_ref, y_ref, dinv_ref):
    deg = jnp.sum(degp_ref[...], axis=0)[:NN]
    dinv = jnp.where(deg > 0.0, lax.rsqrt(jnp.maximum(deg, 1e-12)), 0.0)
    xw = jnp.dot(x_ref[...], w_ref[...], preferred_element_type=jnp.float32)
    y_ref[...] = xw * dinv[:, None]
    dinv_ref[...] = dinv[:, None]


def _k4_body(p_ref, dinv_ref, b1_ref, w2_ref, y2_ref):
    dinv = dinv_ref[...]
    agg = (p_ref[0, :NN, :] + p_ref[1, :NN, :]) * dinv
    h = jnp.maximum(agg + b1_ref[...], 0.0)
    y2_ref[...] = jnp.dot(h, w2_ref[...],
                          preferred_element_type=jnp.float32) * dinv


def _k6_body(p_ref, dinv_ref, b2_ref, batch_ref, wl_ref, bl_ref, out_ref):
    h = (p_ref[0, :NN, :] + p_ref[1, :NN, :]) * dinv_ref[...] + b2_ref[...]
    g = lax.broadcasted_iota(jnp.int32, (NN, GG), 1)
    m = (batch_ref[...] == g).astype(jnp.float32)
    sums = lax.dot_general(m, h, (((0,), (0,)), ((), ())),
                           preferred_element_type=jnp.float32)
    counts = jnp.sum(m, axis=0)
    pooled = sums / jnp.clip(counts, 1.0)[:, None]
    out_ref[...] = jnp.dot(pooled, wl_ref[...],
                           preferred_element_type=jnp.float32) + bl_ref[...]


_k2 = pl.pallas_call(
    _k2_body,
    out_shape=(jax.ShapeDtypeStruct((NN, DH), jnp.float32),
               jax.ShapeDtypeStruct((NN, 1), jnp.float32)))

_k4 = pl.pallas_call(
    _k4_body,
    out_shape=jax.ShapeDtypeStruct((NN, DH), jnp.float32))

_k6 = pl.pallas_call(
    _k6_body,
    out_shape=jax.ShapeDtypeStruct((GG, 64), jnp.float32))


def kernel(x, edge_index, edge_attr, batch, W1, b1, W2, b2, Wl, bl):
    def _split(flat, fill):
        padded = jnp.concatenate(
            [flat, jnp.full((EPAD,), fill, jnp.int32)])
        cut = NS * NCH0 * CHUNK
        a = padded[:cut].reshape(NS, NCH0, CHUNK)
        b = padded[cut:].reshape(NS, NCH1, CHUNK)
        b = jnp.pad(b, ((0, 0), (0, NCHMAX - NCH1), (0, 0)),
                    constant_values=fill)
        return jnp.concatenate([a, b], axis=0)

    src_r = _split(edge_index[0], 0)
    dst_r = _split(edge_index[1], NN)

    degp = _deg_kernel(dst_r)
    y1, dinv = _k2(x, W1, degp)
    p1 = _edge_kernel(y1, src_r, dst_r)
    y2 = _k4(p1, dinv, b1.reshape(1, DH), W2)
    p2 = _edge_kernel(y2, src_r, dst_r)
    logits = _k6(p2, dinv, b2.reshape(1, DH), batch.reshape(NN, 1),
                 Wl, bl.reshape(1, 64))
    return logits


# speedup vs baseline: 1.3133x; 1.3133x over previous
import functools

import jax
import jax.numpy as jnp
from jax import lax
from jax.experimental import pallas as pl
from jax.experimental.pallas import tpu as pltpu
from jax.experimental.pallas import tpu_sc as plsc

NN = 10000
EE = 320000
GG = 128
DH = 128

NC, NS = 2, 16
NW = NC * NS
CHUNK = 128
NCH0 = 79
NCH1 = 79
NCHMAX = NCH0
TOTCH = NS * (NCH0 + NCH1)
EPAD = TOTCH * CHUNK - EE
NPAD = 10240
RPS = NPAD // NS
DEGPAD = 10240

_mesh = plsc.VectorSubcoreMesh(
    core_axis_name="c", subcore_axis_name="s", num_cores=NC, num_subcores=NS)
_sc_params = pltpu.CompilerParams(needs_layout_passes=False)


@functools.partial(
    pl.kernel,
    out_type=jax.ShapeDtypeStruct((NW, DEGPAD), jnp.float32),
    mesh=_mesh,
    compiler_params=_sc_params,
    scratch_types=[
        pltpu.VMEM((NCHMAX, CHUNK), jnp.int32),
        pltpu.VMEM((DEGPAD,), jnp.float32),
    ],
)
def _deg_kernel(dst_hbm, deg_out, dst_v, deg_v):
    c = lax.axis_index("c")
    wid = c * NS + lax.axis_index("s")
    pltpu.sync_copy(dst_hbm.at[wid], dst_v)

    @pl.loop(0, DEGPAD // 16)
    def _zero(i):
        deg_v[pl.ds(i * 16, 16)] = jnp.zeros((16,), jnp.float32)

    ones = jnp.ones((16,), jnp.float32)

    def _hist(nch):
        @pl.loop(0, nch)
        def _row(j):
            @pl.loop(0, CHUNK // 16)
            def _sub(t):
                idx = dst_v[j, pl.ds(t * 16, 16)]
                plsc.addupdate_scatter(deg_v, [idx], ones)

    if NCH0 == NCH1:
        _hist(NCH0)
    else:
        @pl.when(c == 0)
        def _():
            _hist(NCH0)

        @pl.when(c != 0)
        def _():
            _hist(NCH1)

    pltpu.sync_copy(deg_v, deg_out.at[wid])


@functools.partial(
    pl.kernel,
    out_type=jax.ShapeDtypeStruct((NC, NPAD, DH), jnp.float32),
    mesh=_mesh,
    compiler_params=_sc_params,
    scratch_types=[
        pltpu.VMEM((NCHMAX, CHUNK), jnp.int32),
        pltpu.VMEM((NCHMAX, CHUNK), jnp.int32),
        pltpu.VMEM((CHUNK, DH), jnp.float32),
        pltpu.VMEM_SHARED((NPAD, DH), jnp.float32),
        pltpu.SemaphoreType.DMA,
        pltpu.SemaphoreType.DMA,
        pltpu.SemaphoreType.DMA,
    ],
)
def _edge_kernel(y_hbm, src_hbm, dst_hbm, out_hbm,
                 src_v, dst_v, rows_v, agg_sh, gsem, ssem, isem):
    c = lax.axis_index("c")
    s = lax.axis_index("s")
    wid = c * NS + s
    pltpu.sync_copy(src_hbm.at[wid], src_v)
    pltpu.sync_copy(dst_hbm.at[wid], dst_v)

    @pl.loop(0, CHUNK * DH // 16)
    def _zb(i):
        r = i // (DH // 16)
        t = i % (DH // 16)
        rows_v[r, pl.ds(t * 16, 16)] = jnp.zeros((16,), jnp.float32)

    base = s * RPS

    @pl.loop(0, RPS // CHUNK)
    def _zc(i):
        pltpu.sync_copy(rows_v, agg_sh.at[pl.ds(base + i * CHUNK, CHUNK)])

    plsc.subcore_barrier()

    def _run(nch):
        @pl.loop(0, nch)
        def _chunk(j):
            pltpu.async_copy(y_hbm.at[src_v.at[j]], rows_v, gsem).wait()
            pltpu.async_copy(rows_v, agg_sh.at[dst_v.at[j]], ssem,
                             add=True).wait()

    if NCH0 == NCH1:
        _run(NCH0)
    else:
        @pl.when(c == 0)
        def _():
            _run(NCH0)

        @pl.when(c != 0)
        def _():
            _run(NCH1)

    plsc.subcore_barrier()
    pltpu.sync_copy(agg_sh.at[pl.ds(base, RPS)],
                    out_hbm.at[c, pl.ds(base, RPS)])


def _k2_body(x_ref, w_ref, degp_ref, y_ref, dinv_ref):
    deg = jnp.sum(degp_ref[...], axis=0)[:NN]
    dinv = jnp.where(deg > 0.0, lax.rsqrt(jnp.maximum(deg, 1e-12)), 0.0)
    xw = jnp.dot(x_ref[...], w_ref[...], preferred_element_type=jnp.float32)
    y_ref[...] = xw * dinv[:, None]
    dinv_ref[...] = dinv[:, None]


def _k4_body(p_ref, dinv_ref, b1_ref, w2_ref, y2_ref):
    dinv = dinv_ref[...]
    agg = (p_ref[0, :NN, :] + p_ref[1, :NN, :]) * dinv
    h = jnp.maximum(agg + b1_ref[...], 0.0)
    y2_ref[...] = jnp.dot(h, w2_ref[...],
                          preferred_element_type=jnp.float32) * dinv


def _k6_body(p_ref, dinv_ref, b2_ref, batch_ref, wl_ref, bl_ref, out_ref):
    h = (p_ref[0, :NN, :] + p_ref[1, :NN, :]) * dinv_ref[...] + b2_ref[...]
    g = lax.broadcasted_iota(jnp.int32, (NN, GG), 1)
    m = (batch_ref[...] == g).astype(jnp.float32)
    sums = lax.dot_general(m, h, (((0,), (0,)), ((), ())),
                           preferred_element_type=jnp.float32)
    counts = jnp.sum(m, axis=0)
    pooled = sums / jnp.clip(counts, 1.0)[:, None]
    out_ref[...] = jnp.dot(pooled, wl_ref[...],
                           preferred_element_type=jnp.float32) + bl_ref[...]


_k2 = pl.pallas_call(
    _k2_body,
    out_shape=(jax.ShapeDtypeStruct((NN, DH), jnp.float32),
               jax.ShapeDtypeStruct((NN, 1), jnp.float32)))

_k4 = pl.pallas_call(
    _k4_body,
    out_shape=jax.ShapeDtypeStruct((NN, DH), jnp.float32))

_k6 = pl.pallas_call(
    _k6_body,
    out_shape=jax.ShapeDtypeStruct((GG, 64), jnp.float32))


def kernel(x, edge_index, edge_attr, batch, W1, b1, W2, b2, Wl, bl):
    def _split(flat, fill):
        padded = jnp.concatenate(
            [flat, jnp.full((EPAD,), fill, jnp.int32)])
        cut = NS * NCH0 * CHUNK
        a = padded[:cut].reshape(NS, NCH0, CHUNK)
        b = padded[cut:].reshape(NS, NCH1, CHUNK)
        b = jnp.pad(b, ((0, 0), (0, NCHMAX - NCH1), (0, 0)),
                    constant_values=fill)
        return jnp.concatenate([a, b], axis=0)

    src_r = _split(edge_index[0], 0)
    dst_r = _split(edge_index[1], NN)

    degp = _deg_kernel(dst_r)
    y1, dinv = _k2(x, W1, degp)
    p1 = _edge_kernel(y1, src_r, dst_r)
    y2 = _k4(p1, dinv, b1.reshape(1, DH), W2)
    p2 = _edge_kernel(y2, src_r, dst_r)
    logits = _k6(p2, dinv, b2.reshape(1, DH), batch.reshape(NN, 1),
                 Wl, bl.reshape(1, 64))
    return logits


# speedup vs baseline: 2.3361x; 1.7789x over previous
import functools

import jax
import jax.numpy as jnp
from jax import lax
from jax.experimental import pallas as pl
from jax.experimental.pallas import tpu as pltpu
from jax.experimental.pallas import tpu_sc as plsc

NN = 10000
EE = 320000
GG = 128
DH = 128

NC, NS = 2, 16
NW = NC * NS
CHUNK = 128
NCH0 = 79
NCH1 = 79
NCHMAX = NCH0
TOTCH = NS * (NCH0 + NCH1)
EPAD = TOTCH * CHUNK - EE
NPAD = 10240
RPS = NPAD // NS
DEGPAD = 10240

_mesh = plsc.VectorSubcoreMesh(
    core_axis_name="c", subcore_axis_name="s", num_cores=NC, num_subcores=NS)
_sc_params = pltpu.CompilerParams(needs_layout_passes=False)


@functools.partial(
    pl.kernel,
    out_type=jax.ShapeDtypeStruct((NW, DEGPAD), jnp.float32),
    mesh=_mesh,
    compiler_params=_sc_params,
    scratch_types=[
        pltpu.VMEM((NCHMAX, CHUNK), jnp.int32),
        pltpu.VMEM((DEGPAD,), jnp.float32),
    ],
)
def _deg_kernel(dst_hbm, deg_out, dst_v, deg_v):
    c = lax.axis_index("c")
    wid = c * NS + lax.axis_index("s")
    pltpu.sync_copy(dst_hbm.at[wid], dst_v)

    @pl.loop(0, DEGPAD // 16)
    def _zero(i):
        deg_v[pl.ds(i * 16, 16)] = jnp.zeros((16,), jnp.float32)

    ones = jnp.ones((16,), jnp.float32)

    def _hist(nch):
        @pl.loop(0, nch)
        def _row(j):
            @pl.loop(0, CHUNK // 16)
            def _sub(t):
                idx = dst_v[j, pl.ds(t * 16, 16)]
                plsc.addupdate_scatter(deg_v, [idx], ones)

    if NCH0 == NCH1:
        _hist(NCH0)
    else:
        @pl.when(c == 0)
        def _():
            _hist(NCH0)

        @pl.when(c != 0)
        def _():
            _hist(NCH1)

    pltpu.sync_copy(deg_v, deg_out.at[wid])


@functools.partial(
    pl.kernel,
    out_type=jax.ShapeDtypeStruct((NC, NPAD, DH), jnp.float32),
    mesh=_mesh,
    compiler_params=_sc_params,
    scratch_types=[
        pltpu.VMEM((NCHMAX, CHUNK), jnp.int32),
        pltpu.VMEM((NCHMAX, CHUNK), jnp.int32),
        pltpu.VMEM((CHUNK, DH), jnp.float32),
        pltpu.VMEM_SHARED((NPAD, DH), jnp.float32),
        pltpu.SemaphoreType.DMA,
        pltpu.SemaphoreType.DMA,
        pltpu.SemaphoreType.DMA,
    ],
)
def _edge_kernel(y_hbm, src_hbm, dst_hbm, out_hbm,
                 src_v, dst_v, rows_v, agg_sh, gsem, ssem, isem):
    c = lax.axis_index("c")
    s = lax.axis_index("s")
    wid = c * NS + s
    pltpu.sync_copy(src_hbm.at[wid], src_v)
    pltpu.sync_copy(dst_hbm.at[wid], dst_v)

    @pl.loop(0, CHUNK * DH // 16)
    def _zb(i):
        r = i // (DH // 16)
        t = i % (DH // 16)
        rows_v[r, pl.ds(t * 16, 16)] = jnp.zeros((16,), jnp.float32)

    base = s * RPS

    @pl.loop(0, RPS // CHUNK)
    def _zc(i):
        pltpu.sync_copy(rows_v, agg_sh.at[pl.ds(base + i * CHUNK, CHUNK)])

    plsc.subcore_barrier()

    def _run(nch):
        @pl.loop(0, nch)
        def _chunk(j):
            pltpu.async_copy(y_hbm.at[src_v.at[j]], rows_v, gsem).wait()
            pltpu.async_copy(rows_v, agg_sh.at[dst_v.at[j]], ssem,
                             add=True).wait()

    if NCH0 == NCH1:
        _run(NCH0)
    else:
        @pl.when(c == 0)
        def _():
            _run(NCH0)

        @pl.when(c != 0)
        def _():
            _run(NCH1)

    plsc.subcore_barrier()
    pltpu.sync_copy(agg_sh.at[pl.ds(base, RPS)],
                    out_hbm.at[c, pl.ds(base, RPS)])


def _k2_body(x_ref, w_ref, degp_ref, y_ref, dinv_ref):
    deg = jnp.sum(degp_ref[...], axis=0)[:NN]
    dinv = jnp.where(deg > 0.0, lax.rsqrt(jnp.maximum(deg, 1e-12)), 0.0)
    xw = jnp.dot(x_ref[...], w_ref[...], preferred_element_type=jnp.float32)
    y_ref[...] = xw * dinv[:, None]
    dinv_ref[...] = dinv[:, None]


def _k4_body(p_ref, dinv_ref, b1_ref, w2_ref, y2_ref):
    dinv = dinv_ref[...]
    agg = (p_ref[0, :NN, :] + p_ref[1, :NN, :]) * dinv
    h = jnp.maximum(agg + b1_ref[...], 0.0)
    y2_ref[...] = jnp.dot(h, w2_ref[...],
                          preferred_element_type=jnp.float32) * dinv


def _k6_body(p_ref, dinv_ref, b2_ref, batch_ref, wl_ref, bl_ref, out_ref):
    h = (p_ref[0, :NN, :] + p_ref[1, :NN, :]) * dinv_ref[...] + b2_ref[...]
    g = lax.broadcasted_iota(jnp.int32, (NN, GG), 1)
    m = (batch_ref[...] == g).astype(jnp.float32)
    sums = lax.dot_general(m, h, (((0,), (0,)), ((), ())),
                           preferred_element_type=jnp.float32)
    counts = jnp.sum(m, axis=0)
    pooled = sums / jnp.clip(counts, 1.0)[:, None]
    out_ref[...] = jnp.dot(pooled, wl_ref[...],
                           preferred_element_type=jnp.float32) + bl_ref[...]


_k2 = pl.pallas_call(
    _k2_body,
    out_shape=(jax.ShapeDtypeStruct((NN, DH), jnp.float32),
               jax.ShapeDtypeStruct((NN, 1), jnp.float32)))

_k4 = pl.pallas_call(
    _k4_body,
    out_shape=jax.ShapeDtypeStruct((NN, DH), jnp.float32))

_k6 = pl.pallas_call(
    _k6_body,
    out_shape=jax.ShapeDtypeStruct((GG, 64), jnp.float32))


def kernel(x, edge_index, edge_attr, batch, W1, b1, W2, b2, Wl, bl):
    def _split(flat, padvals):
        padded = jnp.concatenate([flat, padvals])
        cut = NS * NCH0 * CHUNK
        a = padded[:cut].reshape(NS, NCH0, CHUNK)
        b = padded[cut:].reshape(NS, NCH1, CHUNK)
        b = jnp.pad(b, ((0, 0), (0, NCHMAX - NCH1), (0, 0)))
        return jnp.concatenate([a, b], axis=0)

    pad_iota = jnp.arange(EPAD, dtype=jnp.int32)
    src_r = _split(edge_index[0], pad_iota % NN)
    dst_r = _split(edge_index[1], NN + pad_iota % (NPAD - NN))

    degp = _deg_kernel(dst_r)
    y1, dinv = _k2(x, W1, degp)
    p1 = _edge_kernel(y1, src_r, dst_r)
    y2 = _k4(p1, dinv, b1.reshape(1, DH), W2)
    p2 = _edge_kernel(y2, src_r, dst_r)
    logits = _k6(p2, dinv, b2.reshape(1, DH), batch.reshape(NN, 1),
                 Wl, bl.reshape(1, 64))
    return logits
